# baseline (speedup 1.0000x reference)
import functools

import jax
import jax.numpy as jnp
from jax import lax
from jax.experimental import pallas as pl
from jax.experimental.pallas import tpu as pltpu
from jax.experimental.pallas import tpu_sc as plsc

N = 10000
D = 128
NPAD = 10240
E = 160000
NSUB = 16
CHUNK = 52
CPT = 193
EPT = CPT * CHUNK
EPAD = EPT * NSUB
NCHUNKS = EPAD // CHUNK
NACC = 10112
ZPT = NACC // NSUB


def _attn_body(q_ref, k_ref, bias_ref, u_ref):
    q = q_ref[0]
    kb = k_ref[0].astype(jnp.bfloat16)
    s = lax.dot_general(q.astype(jnp.bfloat16), kb, (((1,), (1,)), ((), ())),
                        preferred_element_type=jnp.float32)
    s = s + bias_ref[...]
    m = jnp.max(s, axis=1, keepdims=True)
    p = jnp.exp(s - m)
    denom = jnp.sum(p, axis=1, keepdims=True)
    o = lax.dot_general(p.astype(jnp.bfloat16), kb, (((1,), (0,)), ((), ())),
                        preferred_element_type=jnp.float32)
    u_ref[0] = q - o / denom


def _cross_attention(hp, bias):
    bq = 256
    return pl.pallas_call(
        _attn_body,
        grid=(2, NPAD // bq),
        in_specs=[
            pl.BlockSpec((1, bq, D), lambda g, i: (g, i, 0)),
            pl.BlockSpec((1, NPAD, D), lambda g, i: (1 - g, 0, 0)),
            pl.BlockSpec((1, NPAD), lambda g, i: (0, 0)),
        ],
        out_specs=pl.BlockSpec((1, bq, D), lambda g, i: (g, i, 0)),
        out_shape=jax.ShapeDtypeStruct((2, NPAD, D), jnp.float32),
    )(hp, hp, bias)


def _prep_body(hp_ref, w_ref, b_ref, o_ref):
    x = hp_ref[0]
    o_ref[...] = lax.dot_general(x, w_ref[...], (((1,), (0,)), ((), ())),
                                 preferred_element_type=jnp.float32) + b_ref[0]


def _prep_tables(hp, w2, b2):
    blk = 1024
    nb = NPAD // blk
    return pl.pallas_call(
        _prep_body,
        grid=(4, nb),
        in_specs=[
            pl.BlockSpec((1, blk, D), lambda q, i: (q // 2, i, 0)),
            pl.BlockSpec((D, D), lambda q, i: (0, q % 2)),
            pl.BlockSpec((1, 1, D), lambda q, i: (q % 2, 0, 0)),
        ],
        out_specs=pl.BlockSpec((blk, D), lambda q, i: (q * nb + i, 0)),
        out_shape=jax.ShapeDtypeStruct((4 * NPAD, D), jnp.float32),
    )(hp, w2, b2)


def _small_matmul_body(x_ref, w_ref, o_ref):
    o_ref[...] = lax.dot_general(x_ref[...], w_ref[...],
                                 (((1,), (0,)), ((), ())),
                                 preferred_element_type=jnp.float32)


def _edge_table(edge_embed, wc):
    return pl.pallas_call(
        _small_matmul_body,
        out_shape=jax.ShapeDtypeStruct((16, D), jnp.float32),
    )(edge_embed, wc)


_SC_MESH = plsc.VectorSubcoreMesh(core_axis_name="c", subcore_axis_name="s")


def _propagate_sc(tbl, ctbl, idx, zeros):
    @functools.partial(
        pl.kernel,
        out_type=jax.ShapeDtypeStruct((2, NACC, D), jnp.float32),
        mesh=_SC_MESH,
        scratch_types=[
            pltpu.VMEM((2, 4, CHUNK), jnp.int32),
            pltpu.VMEM((2, CHUNK), jnp.int32),
            pltpu.VMEM((2, CHUNK, D), jnp.float32),
            pltpu.VMEM((2, CHUNK, D), jnp.float32),
            pltpu.VMEM((2, CHUNK, D), jnp.float32),
            pltpu.VMEM((16, D), jnp.float32),
            pltpu.VMEM_SHARED((NACC, D), jnp.float32),
            pltpu.SemaphoreType.DMA,
            pltpu.SemaphoreType.DMA,
            pltpu.SemaphoreType.DMA,
            pltpu.SemaphoreType.DMA,
        ],
    )
    def k(t_hbm, c_hbm, idx_hbm, z_hbm, out_hbm,
          ibuf, sbidx, arows, brows, rrows, c_local, acc, g0, g1, s0, s1):
        cid = lax.axis_index("c")
        sid = lax.axis_index("s")
        gsem = (g0, g1)
        ssem = (s0, s1)
        windows = [(w, 0) for w in range(0, CHUNK - 15, 16)]
        tail = CHUNK % 16
        if tail:
            windows.append((CHUNK - 16, 16 - tail))

        def issue_gathers(kk, p):
            pltpu.sync_copy(idx_hbm.at[cid, sid * CPT + kk], ibuf.at[p])
            pltpu.async_copy(t_hbm.at[ibuf.at[p, 0]], arows.at[p], gsem[p])
            pltpu.async_copy(t_hbm.at[ibuf.at[p, 1]], brows.at[p], gsem[p])

        def wait_gathers(p):
            pltpu.make_async_copy(t_hbm.at[ibuf.at[p, 0]], arows.at[p],
                                  gsem[p]).wait()
            pltpu.make_async_copy(t_hbm.at[ibuf.at[p, 1]], brows.at[p],
                                  gsem[p]).wait()

        def wait_scatter(p):
            pltpu.make_async_copy(rrows.at[p], acc.at[sbidx.at[p]],
                                  ssem[p]).wait()

        issue_gathers(0, 0)
        pltpu.sync_copy(c_hbm, c_local)
        pltpu.sync_copy(z_hbm.at[pl.ds(sid * ZPT, ZPT)],
                        acc.at[pl.ds(sid * ZPT, ZPT)])
        plsc.subcore_barrier()

        def body(kk, p):
            wait_gathers(p)

            @pl.when(kk + 1 < CPT)
            def _():
                issue_gathers(kk + 1, 1 - p)

            @pl.when(kk >= 2)
            def _():
                wait_scatter(p)

            for w, _skip in windows:
                sbidx[p, pl.ds(w, 16)] = ibuf[p, 3, pl.ds(w, 16)]

            for w, skip in windows:
                vattr = ibuf[p, 2, pl.ds(w, 16)]
                for l in range(skip, 16):
                    e = w + l
                    ae = vattr[l]
                    for j in range(D // 16):
                        sl = pl.ds(j * 16, 16)
                        v = arows[p, e, sl] + brows[p, e, sl] + c_local[ae, sl]
                        rrows[p, e, sl] = jnp.maximum(v, 0.0)

            pltpu.async_copy(rrows.at[p], acc.at[sbidx.at[p]], ssem[p],
                             add=True)

        @pl.loop(0, CPT)
        def _(kk):
            @pl.when(kk % 2 == 0)
            def _():
                body(kk, 0)

            @pl.when(kk % 2 == 1)
            def _():
                body(kk, 1)

        wait_scatter((CPT - 1) % 2)
        wait_scatter((CPT - 2) % 2)
        plsc.subcore_barrier()
        pltpu.sync_copy(acc.at[pl.ds(sid * ZPT, ZPT)],
                        out_hbm.at[cid, pl.ds(sid * ZPT, ZPT)])

    return k(tbl, ctbl, idx, zeros)


def _build_idx(edge_index, edge_attr, g):
    src = edge_index[0]
    dst = edge_index[1]
    attr = edge_attr[:, 0]
    dstp = jnp.pad(dst, (0, EPAD - E), constant_values=N)
    srcp = jnp.pad(src, (0, EPAD - E), constant_values=N)
    attrp = jnp.pad(attr, (0, EPAD - E))
    rows = jnp.stack([dstp + (2 * g) * NPAD,
                      srcp + (2 * g + 1) * NPAD,
                      attrp,
                      dstp])
    return rows.reshape(4, NCHUNKS, CHUNK).transpose(1, 0, 2)


_BGRU = 400
_NBGRU = N // _BGRU


def _tail_body(m_ref, u_ref, h_ref, wim_ref, wiu_ref, whh_ref,
               bih_ref, bhh_ref, wg_ref, bg_ref, hg_ref, num_ref, den_ref):
    i = pl.program_id(1)

    mm = m_ref[0]
    u = u_ref[0]
    h = h_ref[0]
    gi = (lax.dot_general(mm, wim_ref[...], (((1,), (0,)), ((), ())),
                          preferred_element_type=jnp.float32)
          + lax.dot_general(u, wiu_ref[...], (((1,), (0,)), ((), ())),
                            preferred_element_type=jnp.float32)
          + bih_ref[...])
    gh = lax.dot_general(h, whh_ref[...], (((1,), (0,)), ((), ())),
                         preferred_element_type=jnp.float32) + bhh_ref[...]
    r = jax.nn.sigmoid(gi[:, :D] + gh[:, :D])
    z = jax.nn.sigmoid(gi[:, D:2 * D] + gh[:, D:2 * D])
    n = jnp.tanh(gi[:, 2 * D:] + r * gh[:, 2 * D:])
    hn = (1.0 - z) * n + z * h

    logit = jnp.sum(hn * wg_ref[...], axis=1, keepdims=True) + bg_ref[...]
    e = jnp.exp(jax.nn.sigmoid(logit))
    num = jnp.sum(e * hn, axis=0, keepdims=True)
    den = jnp.sum(e, axis=0, keepdims=True)

    @pl.when(i == 0)
    def _():
        num_ref[...] = jnp.zeros_like(num_ref)
        den_ref[...] = jnp.zeros_like(den_ref)

    num_ref[...] += num
    den_ref[...] += den

    @pl.when(i == _NBGRU - 1)
    def _():
        hg_ref[0] = num_ref[...] / den_ref[...]


def _gru_global_tail(m, u, hp, wim, wiu, whh, bih2, bhh2, wg, bg2):
    return pl.pallas_call(
        _tail_body,
        grid=(2, _NBGRU),
        in_specs=[
            pl.BlockSpec((1, _BGRU, D), lambda g, i: (g, i, 0)),
            pl.BlockSpec((1, _BGRU, D), lambda g, i: (g, i, 0)),
            pl.BlockSpec((1, _BGRU, D), lambda g, i: (g, i, 0)),
            pl.BlockSpec((D, 3 * D), lambda g, i: (0, 0)),
            pl.BlockSpec((D, 3 * D), lambda g, i: (0, 0)),
            pl.BlockSpec((D, 3 * D), lambda g, i: (0, 0)),
            pl.BlockSpec((1, 3 * D), lambda g, i: (0, 0)),
            pl.BlockSpec((1, 3 * D), lambda g, i: (0, 0)),
            pl.BlockSpec((1, D), lambda g, i: (0, 0)),
            pl.BlockSpec((1, 1), lambda g, i: (0, 0)),
        ],
        out_specs=pl.BlockSpec((1, 1, D), lambda g, i: (g, 0, 0)),
        out_shape=jax.ShapeDtypeStruct((2, 1, D), jnp.float32),
        scratch_shapes=[
            pltpu.VMEM((1, D), jnp.float32),
            pltpu.VMEM((1, 1), jnp.float32),
        ],
    )(m, u, hp, wim, wiu, whh, bih2, bhh2, wg, bg2)


@jax.jit
def kernel(x1, x2, edge_index1, edge_index2, edge_attr1, edge_attr2,
           embed, edge_embed, Wm, bm, Wih, Whh, bih, bhh, Wg, bg):
    nodes = jnp.stack([x1[:, 0], x2[:, 0]])
    hp = jnp.take(embed, nodes, axis=0)
    hp = jnp.pad(hp, ((0, 0), (0, NPAD - N), (0, 0)))

    w2 = jnp.concatenate([Wm[:, :D].T, Wm[:, D:2 * D].T], axis=1)
    b2 = jnp.stack([bm, jnp.zeros_like(bm)])[:, None, :]
    tbl = _prep_tables(hp, w2, b2)
    ctbl = _edge_table(edge_embed, Wm[:, 2 * D:].T)

    idx = jnp.stack([_build_idx(edge_index1, edge_attr1, 0),
                     _build_idx(edge_index2, edge_attr2, 1)])
    zeros = jnp.zeros((NACC, D), jnp.float32)
    m = _propagate_sc(tbl, ctbl, idx, zeros)

    bias = jnp.where(jnp.arange(NPAD) < N, 0.0, -1e30)[None, :].astype(jnp.float32)
    u = _cross_attention(hp, bias)

    wim = Wih[:, :D].T
    wiu = Wih[:, D:].T
    hg = _gru_global_tail(m, u, hp, wim, wiu, Whh.T,
                          bih[None, :], bhh[None, :], Wg, bg[None, :])
    return (hg[0], hg[1])


# speedup vs baseline: 1.6393x; 1.6393x over previous
import functools

import jax
import jax.numpy as jnp
from jax import lax
from jax.experimental import pallas as pl
from jax.experimental.pallas import tpu as pltpu
from jax.experimental.pallas import tpu_sc as plsc

N = 10000
D = 128
NPAD = 10240
E = 160000
NSUB = 16
CHUNK = 80
CPT = 125
EPT = CPT * CHUNK
EPAD = EPT * NSUB
NCHUNKS = EPAD // CHUNK
NACC = 10112
ZPT = NACC // NSUB


def _attn_body(q_ref, k_ref, bias_ref, u_ref):
    q = q_ref[0]
    kb = k_ref[0].astype(jnp.bfloat16)
    s = lax.dot_general(q.astype(jnp.bfloat16), kb, (((1,), (1,)), ((), ())),
                        preferred_element_type=jnp.float32)
    s = s + bias_ref[...]
    m = jnp.max(s, axis=1, keepdims=True)
    p = jnp.exp(s - m)
    denom = jnp.sum(p, axis=1, keepdims=True)
    o = lax.dot_general(p.astype(jnp.bfloat16), kb, (((1,), (0,)), ((), ())),
                        preferred_element_type=jnp.float32)
    u_ref[0] = q - o / denom


def _cross_attention(hp, bias):
    bq = 256
    return pl.pallas_call(
        _attn_body,
        grid=(2, NPAD // bq),
        in_specs=[
            pl.BlockSpec((1, bq, D), lambda g, i: (g, i, 0)),
            pl.BlockSpec((1, NPAD, D), lambda g, i: (1 - g, 0, 0)),
            pl.BlockSpec((1, NPAD), lambda g, i: (0, 0)),
        ],
        out_specs=pl.BlockSpec((1, bq, D), lambda g, i: (g, i, 0)),
        out_shape=jax.ShapeDtypeStruct((2, NPAD, D), jnp.float32),
    )(hp, hp, bias)


def _prep_body(hp_ref, w_ref, b_ref, o_ref):
    x = hp_ref[0]
    o_ref[...] = lax.dot_general(x, w_ref[...], (((1,), (0,)), ((), ())),
                                 preferred_element_type=jnp.float32) + b_ref[0]


def _prep_tables(hp, w2, b2):
    blk = 1024
    nb = NPAD // blk
    return pl.pallas_call(
        _prep_body,
        grid=(4, nb),
        in_specs=[
            pl.BlockSpec((1, blk, D), lambda q, i: (q // 2, i, 0)),
            pl.BlockSpec((D, D), lambda q, i: (0, q % 2)),
            pl.BlockSpec((1, 1, D), lambda q, i: (q % 2, 0, 0)),
        ],
        out_specs=pl.BlockSpec((blk, D), lambda q, i: (q * nb + i, 0)),
        out_shape=jax.ShapeDtypeStruct((4 * NPAD, D), jnp.float32),
    )(hp, w2, b2)


def _small_matmul_body(x_ref, w_ref, o_ref):
    o_ref[...] = lax.dot_general(x_ref[...], w_ref[...],
                                 (((1,), (0,)), ((), ())),
                                 preferred_element_type=jnp.float32)


def _edge_table(edge_embed, wc):
    return pl.pallas_call(
        _small_matmul_body,
        out_shape=jax.ShapeDtypeStruct((16, D), jnp.float32),
    )(edge_embed, wc)


_SC_MESH = plsc.VectorSubcoreMesh(core_axis_name="c", subcore_axis_name="s")


def _propagate_sc(tbl, ctbl, idx, zeros):
    @functools.partial(
        pl.kernel,
        out_type=jax.ShapeDtypeStruct((2, NACC, D), jnp.float32),
        mesh=_SC_MESH,
        scratch_types=[
            pltpu.VMEM((2, 4, CHUNK), jnp.int32),
            pltpu.VMEM((2, CHUNK, D), jnp.float32),
            pltpu.VMEM((2, CHUNK, D), jnp.float32),
            pltpu.VMEM((16, D), jnp.float32),
            pltpu.VMEM_SHARED((NACC, D), jnp.float32),
            pltpu.SemaphoreType.DMA,
            pltpu.SemaphoreType.DMA,
            pltpu.SemaphoreType.DMA,
            pltpu.SemaphoreType.DMA,
        ],
    )
    def k(t_hbm, c_hbm, idx_hbm, z_hbm, out_hbm,
          ibuf, arows, brows, c_local, acc, g0, g1, s0, s1):
        cid = lax.axis_index("c")
        sid = lax.axis_index("s")
        gsem = (g0, g1)
        ssem = (s0, s1)

        def issue_gathers(kk, p):
            pltpu.sync_copy(idx_hbm.at[cid, sid * CPT + kk], ibuf.at[p])
            pltpu.async_copy(t_hbm.at[ibuf.at[p, 0]], arows.at[p], gsem[p])
            pltpu.async_copy(t_hbm.at[ibuf.at[p, 1]], brows.at[p], gsem[p])

        def wait_gathers(p):
            pltpu.make_async_copy(t_hbm.at[ibuf.at[p, 0]], arows.at[p],
                                  gsem[p]).wait()
            pltpu.make_async_copy(t_hbm.at[ibuf.at[p, 1]], brows.at[p],
                                  gsem[p]).wait()

        def wait_scatter(p):
            pltpu.make_async_copy(arows.at[p], acc.at[ibuf.at[p, 3]],
                                  ssem[p]).wait()

        issue_gathers(0, 0)
        pltpu.sync_copy(c_hbm, c_local)
        pltpu.sync_copy(z_hbm.at[pl.ds(sid * ZPT, ZPT)],
                        acc.at[pl.ds(sid * ZPT, ZPT)])
        plsc.subcore_barrier()

        def body(kk, p):
            q = 1 - p
            wait_gathers(p)

            @pl.when(kk >= 1)
            def _():
                wait_scatter(q)

            @pl.when(kk + 1 < CPT)
            def _():
                issue_gathers(kk + 1, q)

            @pl.loop(0, CHUNK // 16)
            def _(g):
                vattr = ibuf[p, 2, pl.ds(g * 16, 16)]
                for l in range(16):
                    e = g * 16 + l
                    ae = vattr[l]
                    for j in range(D // 16):
                        sl = pl.ds(j * 16, 16)
                        v = arows[p, e, sl] + brows[p, e, sl] + c_local[ae, sl]
                        arows[p, e, sl] = jnp.maximum(v, 0.0)

            pltpu.async_copy(arows.at[p], acc.at[ibuf.at[p, 3]], ssem[p],
                             add=True)

        @pl.loop(0, CPT)
        def _(kk):
            @pl.when(kk % 2 == 0)
            def _():
                body(kk, 0)

            @pl.when(kk % 2 == 1)
            def _():
                body(kk, 1)

        wait_scatter((CPT - 1) % 2)
        plsc.subcore_barrier()
        pltpu.sync_copy(acc.at[pl.ds(sid * ZPT, ZPT)],
                        out_hbm.at[cid, pl.ds(sid * ZPT, ZPT)])

    return k(tbl, ctbl, idx, zeros)


def _build_idx(edge_index, edge_attr, g):
    src = edge_index[0]
    dst = edge_index[1]
    attr = edge_attr[:, 0]
    dstp = jnp.pad(dst, (0, EPAD - E), constant_values=N)
    srcp = jnp.pad(src, (0, EPAD - E), constant_values=N)
    attrp = jnp.pad(attr, (0, EPAD - E))
    rows = jnp.stack([dstp + (2 * g) * NPAD,
                      srcp + (2 * g + 1) * NPAD,
                      attrp,
                      dstp])
    return rows.reshape(4, NCHUNKS, CHUNK).transpose(1, 0, 2)


_BGRU = 400
_NBGRU = N // _BGRU


def _tail_body(m_ref, u_ref, h_ref, wim_ref, wiu_ref, whh_ref,
               bih_ref, bhh_ref, wg_ref, bg_ref, hg_ref, num_ref, den_ref):
    i = pl.program_id(1)

    mm = m_ref[0]
    u = u_ref[0]
    h = h_ref[0]
    gi = (lax.dot_general(mm, wim_ref[...], (((1,), (0,)), ((), ())),
                          preferred_element_type=jnp.float32)
          + lax.dot_general(u, wiu_ref[...], (((1,), (0,)), ((), ())),
                            preferred_element_type=jnp.float32)
          + bih_ref[...])
    gh = lax.dot_general(h, whh_ref[...], (((1,), (0,)), ((), ())),
                         preferred_element_type=jnp.float32) + bhh_ref[...]
    r = jax.nn.sigmoid(gi[:, :D] + gh[:, :D])
    z = jax.nn.sigmoid(gi[:, D:2 * D] + gh[:, D:2 * D])
    n = jnp.tanh(gi[:, 2 * D:] + r * gh[:, 2 * D:])
    hn = (1.0 - z) * n + z * h

    logit = jnp.sum(hn * wg_ref[...], axis=1, keepdims=True) + bg_ref[...]
    e = jnp.exp(jax.nn.sigmoid(logit))
    num = jnp.sum(e * hn, axis=0, keepdims=True)
    den = jnp.sum(e, axis=0, keepdims=True)

    @pl.when(i == 0)
    def _():
        num_ref[...] = jnp.zeros_like(num_ref)
        den_ref[...] = jnp.zeros_like(den_ref)

    num_ref[...] += num
    den_ref[...] += den

    @pl.when(i == _NBGRU - 1)
    def _():
        hg_ref[0] = num_ref[...] / den_ref[...]


def _gru_global_tail(m, u, hp, wim, wiu, whh, bih2, bhh2, wg, bg2):
    return pl.pallas_call(
        _tail_body,
        grid=(2, _NBGRU),
        in_specs=[
            pl.BlockSpec((1, _BGRU, D), lambda g, i: (g, i, 0)),
            pl.BlockSpec((1, _BGRU, D), lambda g, i: (g, i, 0)),
            pl.BlockSpec((1, _BGRU, D), lambda g, i: (g, i, 0)),
            pl.BlockSpec((D, 3 * D), lambda g, i: (0, 0)),
            pl.BlockSpec((D, 3 * D), lambda g, i: (0, 0)),
            pl.BlockSpec((D, 3 * D), lambda g, i: (0, 0)),
            pl.BlockSpec((1, 3 * D), lambda g, i: (0, 0)),
            pl.BlockSpec((1, 3 * D), lambda g, i: (0, 0)),
            pl.BlockSpec((1, D), lambda g, i: (0, 0)),
            pl.BlockSpec((1, 1), lambda g, i: (0, 0)),
        ],
        out_specs=pl.BlockSpec((1, 1, D), lambda g, i: (g, 0, 0)),
        out_shape=jax.ShapeDtypeStruct((2, 1, D), jnp.float32),
        scratch_shapes=[
            pltpu.VMEM((1, D), jnp.float32),
            pltpu.VMEM((1, 1), jnp.float32),
        ],
    )(m, u, hp, wim, wiu, whh, bih2, bhh2, wg, bg2)


@jax.jit
def kernel(x1, x2, edge_index1, edge_index2, edge_attr1, edge_attr2,
           embed, edge_embed, Wm, bm, Wih, Whh, bih, bhh, Wg, bg):
    nodes = jnp.stack([x1[:, 0], x2[:, 0]])
    hp = jnp.take(embed, nodes, axis=0)
    hp = jnp.pad(hp, ((0, 0), (0, NPAD - N), (0, 0)))

    w2 = jnp.concatenate([Wm[:, :D].T, Wm[:, D:2 * D].T], axis=1)
    b2 = jnp.stack([bm, jnp.zeros_like(bm)])[:, None, :]
    tbl = _prep_tables(hp, w2, b2)
    ctbl = _edge_table(edge_embed, Wm[:, 2 * D:].T)

    idx = jnp.stack([_build_idx(edge_index1, edge_attr1, 0),
                     _build_idx(edge_index2, edge_attr2, 1)])
    zeros = jnp.zeros((NACC, D), jnp.float32)
    m = _propagate_sc(tbl, ctbl, idx, zeros)

    bias = jnp.where(jnp.arange(NPAD) < N, 0.0, -1e30)[None, :].astype(jnp.float32)
    u = _cross_attention(hp, bias)

    wim = Wih[:, :D].T
    wiu = Wih[:, D:].T
    hg = _gru_global_tail(m, u, hp, wim, wiu, Whh.T,
                          bih[None, :], bhh[None, :], Wg, bg[None, :])
    return (hg[0], hg[1])
